# Initial kernel scaffold; baseline (speedup 1.0000x reference)
import jax
import jax.numpy as jnp
from jax.experimental import pallas as pl


def kernel(x, edge_index, W1, b1, bn_g, bn_b, W2, b2, ln_g, ln_b, Wl, bl):
    raise NotImplementedError("write your pallas kernel here")


# speedup vs baseline: 39.4715x; 39.4715x over previous
import functools

import jax
import jax.numpy as jnp
from jax import lax
from jax.experimental import pallas as pl
from jax.experimental.pallas import tpu as pltpu
from jax.experimental.pallas import tpu_sc as plsc

NC = 2
NS = 16
NW = NC * NS
LANES = 16

CHUNK = 128


def _wid():
    return lax.axis_index("s") * NC + lax.axis_index("c")


def _sc_rows_body(n_pad, d, a_hbm, b_hbm, src_hbm, dst_hbm,
                  ones_hbm, zdeg_hbm, out1_hbm, degp_hbm,
                  idxs, idxd, bufa, bufb, ones_v, shared_deg, sem_a, sem_b):
    c = lax.axis_index("c")
    s = lax.axis_index("s")
    wid = _wid()
    rows_tile = n_pad // NS

    pltpu.sync_copy(ones_hbm, ones_v)

    if True:
        pltpu.sync_copy(zdeg_hbm, shared_deg.at[pl.ds(s * rows_tile, rows_tile)])
        plsc.subcore_barrier()

        total_chunks = out1_hbm.shape[0] // CHUNK
        nch = (total_chunks - wid + NW - 1) // NW

        def body(j, carry):
            cid = wid + j * NW
            base = cid * CHUNK
            pltpu.sync_copy(src_hbm.at[pl.ds(base, CHUNK)], idxs)
            pltpu.sync_copy(dst_hbm.at[pl.ds(base, CHUNK)], idxd)
            ca = pltpu.async_copy(a_hbm.at[idxs], bufa, sem_a)
            cb = pltpu.async_copy(b_hbm.at[idxd], bufb, sem_b)
            ca.wait()
            cb.wait()

            def addrow(r, carry2):
                for q in range(8):
                    sl = pl.ds(q * LANES, LANES)
                    bufa[r, sl] = bufa[r, sl] + bufb[r, sl]
                return carry2

            lax.fori_loop(0, CHUNK, addrow, 0)
            pltpu.sync_copy(bufa, out1_hbm.at[pl.ds(base, CHUNK)])
            pltpu.sync_copy(ones_v, shared_deg.at[idxd], add=True)
            return carry

        lax.fori_loop(0, nch, body, 0)
        plsc.subcore_barrier()
        sl = pl.ds(s * rows_tile, rows_tile)
        pltpu.sync_copy(shared_deg.at[sl], degp_hbm.at[c].at[sl])


def _sc_agg_body(n_pad, d, t_hbm, src_hbm, dst_hbm, zrow_hbm, g_hbm,
                 idxs, idxd, rows, shared_g, sem_g):
    c = lax.axis_index("c")
    s = lax.axis_index("s")
    wid = _wid()
    rows_tile = n_pad // NS

    if True:
        pltpu.sync_copy(zrow_hbm, shared_g.at[pl.ds(s * rows_tile, rows_tile)])
        plsc.subcore_barrier()

        total_chunks = src_hbm.shape[0] // CHUNK
        nch = (total_chunks - wid + NW - 1) // NW

        def body(j, carry):
            cid = wid + j * NW
            base = cid * CHUNK
            pltpu.sync_copy(src_hbm.at[pl.ds(base, CHUNK)], idxs)
            pltpu.sync_copy(dst_hbm.at[pl.ds(base, CHUNK)], idxd)
            pltpu.async_copy(t_hbm.at[idxs], rows, sem_g).wait()
            pltpu.sync_copy(rows, shared_g.at[idxd], add=True)
            return carry

        lax.fori_loop(0, nch, body, 0)
        plsc.subcore_barrier()
        sl = pl.ds(s * rows_tile, rows_tile)
        pltpu.sync_copy(shared_g.at[sl], g_hbm.at[c].at[sl])


def _k_ab(x_ref, w1at_ref, w1bt_ref, a_ref, b_ref):
    xb = x_ref[...]
    a_ref[...] = jnp.dot(xb, w1at_ref[...], preferred_element_type=jnp.float32)
    b_ref[...] = jnp.dot(xb, w1bt_ref[...], preferred_element_type=jnp.float32)


def _k_dinv(degp_ref, dinv_ref):
    d = degp_ref[0] + degp_ref[1] + 1.0
    dinv_ref[...] = lax.rsqrt(d)


def _k_t1(h0_ref, dinv_ref, t1_ref):
    t1_ref[...] = h0_ref[...] * dinv_ref[...]


def _k_low1(t1_ref, g_ref, dinv_ref, raw_ref, out_ref):
    del raw_ref
    out_ref[...] = dinv_ref[...] * (t1_ref[...] + g_ref[0] + g_ref[1])


def _k_stats(nblocks, nrows, out1_ref, bng_ref, bnb_ref, ab_ref, acc):
    i = pl.program_id(0)
    blk = out1_ref[...]
    ps = jnp.sum(blk, axis=0, keepdims=True)
    pq = jnp.sum(blk * blk, axis=0, keepdims=True)

    @pl.when(i == 0)
    def _():
        acc[0:1] = ps
        acc[1:2] = pq

    @pl.when(i > 0)
    def _():
        acc[0:1] = acc[0:1] + ps
        acc[1:2] = acc[1:2] + pq

    @pl.when(i == nblocks - 1)
    def _():
        inv_n = 1.0 / nrows
        mu = acc[0:1] * inv_n
        ex2 = acc[1:2] * inv_n
        var = ex2 - mu * mu
        alpha = bng_ref[...] * lax.rsqrt(var + 1e-5)
        beta = bnb_ref[...] - mu * alpha
        ab_ref[0:1] = alpha
        ab_ref[1:2] = beta


def _k_h2low(out1_ref, ab_ref, w2t_ref, dinv_ref, t2_ref):
    z = jnp.maximum(out1_ref[...] * ab_ref[0:1] + ab_ref[1:2], 0.0)
    h2 = jnp.dot(z, w2t_ref[...], preferred_element_type=jnp.float32)
    t2_ref[...] = h2 * dinv_ref[...]


def _ln_head(o2, lng_ref, lnb_ref, wlt_ref, bl_ref):
    mu = jnp.mean(o2, axis=1, keepdims=True)
    cc = o2 - mu
    var = jnp.mean(cc * cc, axis=1, keepdims=True)
    y = jnp.maximum(cc * lax.rsqrt(var + 1e-5) * lng_ref[...] + lnb_ref[...],
                    0.0)
    return jnp.dot(y, wlt_ref[...], preferred_element_type=jnp.float32) \
        + bl_ref[0, 0]


def _k_phase2(out1_ref, ab_ref, w2t_ref, b2_ref, lng_ref, lnb_ref, wlt_ref,
              bl_ref, out_ref):
    z = jnp.maximum(out1_ref[...] * ab_ref[0:1] + ab_ref[1:2], 0.0)
    h2 = jnp.dot(z, w2t_ref[...], preferred_element_type=jnp.float32)
    o2 = h2 + b2_ref[...]
    out_ref[...] = _ln_head(o2, lng_ref, lnb_ref, wlt_ref, bl_ref)


def _k_final(t2_ref, g_ref, dinv_ref, b2_ref, lng_ref, lnb_ref, wlt_ref,
             bl_ref, out_ref):
    o2 = dinv_ref[...] * (t2_ref[...] + g_ref[0] + g_ref[1]) + b2_ref[...]
    out_ref[...] = _ln_head(o2, lng_ref, lnb_ref, wlt_ref, bl_ref)


def kernel(x, edge_index, W1, b1, bn_g, bn_b, W2, b2, ln_g, ln_b, Wl, bl):
    del b1
    n, d = x.shape
    e = edge_index.shape[1]
    f32 = jnp.float32
    src = edge_index[0].astype(jnp.int32)
    dst = edge_index[1].astype(jnp.int32)

    w1at = W1[:, :d].T
    w1bt = W1[:, d:].T
    w2t = W2.T
    wlt = Wl.T
    bng2 = bn_g.reshape(1, d)
    bnb2 = bn_b.reshape(1, d)
    b22 = b2.reshape(1, d)
    lng2 = ln_g.reshape(1, d)
    lnb2 = ln_b.reshape(1, d)
    bl2 = bl.reshape(1, 1)

    npad = ((n + 8 * NS - 1) // (8 * NS)) * (8 * NS)
    rows_tile = npad // NS
    ones_d = jnp.ones((CHUNK, d), f32)
    zrow = jnp.zeros((rows_tile, d), f32)

    mesh = plsc.VectorSubcoreMesh(core_axis_name="c", subcore_axis_name="s")

    nlb = n // 1000
    a_mat, b_mat = pl.pallas_call(
        _k_ab,
        grid=(nlb,),
        in_specs=[
            pl.BlockSpec((1000, d), lambda i: (i, 0)),
            pl.BlockSpec((d, d), lambda i: (0, 0)),
            pl.BlockSpec((d, d), lambda i: (0, 0)),
        ],
        out_specs=[
            pl.BlockSpec((1000, d), lambda i: (i, 0)),
            pl.BlockSpec((1000, d), lambda i: (i, 0)),
        ],
        out_shape=[jax.ShapeDtypeStruct((n, d), f32),
                   jax.ShapeDtypeStruct((n, d), f32)],
    )(x, w1at, w1bt)

    sc_rows = functools.partial(
        pl.kernel,
        out_type=(jax.ShapeDtypeStruct((e, d), f32),
                  jax.ShapeDtypeStruct((NC, npad, d), f32)),
        mesh=mesh,
        scratch_types=[
            pltpu.VMEM((CHUNK,), jnp.int32),
            pltpu.VMEM((CHUNK,), jnp.int32),
            pltpu.VMEM((CHUNK, d), f32),
            pltpu.VMEM((CHUNK, d), f32),
            pltpu.VMEM((CHUNK, d), f32),
            pltpu.VMEM_SHARED((npad, d), f32),
            pltpu.SemaphoreType.DMA,
            pltpu.SemaphoreType.DMA,
        ],
    )(functools.partial(_sc_rows_body, npad, d))
    out1raw, degp = sc_rows(a_mat, b_mat, src, dst, ones_d, zrow)

    dinv8 = pl.pallas_call(
        _k_dinv,
        grid=(1,),
        in_specs=[pl.BlockSpec((NC, npad, d), lambda i: (0, 0, 0))],
        out_specs=pl.BlockSpec((npad, d), lambda i: (0, 0)),
        out_shape=jax.ShapeDtypeStruct((npad, d), f32),
    )(degp)
    dinv = dinv8[:, :1]

    t1 = pl.pallas_call(
        _k_t1,
        grid=(nlb,),
        in_specs=[
            pl.BlockSpec((1000, d), lambda i: (i, 0)),
            pl.BlockSpec((1000, 1), lambda i: (i, 0)),
        ],
        out_specs=pl.BlockSpec((1000, d), lambda i: (i, 0)),
        out_shape=jax.ShapeDtypeStruct((n, d), f32),
    )(out1raw, dinv)

    sc_agg = functools.partial(
        pl.kernel,
        out_type=jax.ShapeDtypeStruct((NC, npad, d), f32),
        mesh=mesh,
        scratch_types=[
            pltpu.VMEM((CHUNK,), jnp.int32),
            pltpu.VMEM((CHUNK,), jnp.int32),
            pltpu.VMEM((CHUNK, d), f32),
            pltpu.VMEM_SHARED((npad, d), f32),
            pltpu.SemaphoreType.DMA,
        ],
    )(functools.partial(_sc_agg_body, npad, d))
    g1 = sc_agg(t1, src, dst, zrow)

    out1 = pl.pallas_call(
        _k_low1,
        grid=(nlb,),
        in_specs=[
            pl.BlockSpec((1000, d), lambda i: (i, 0)),
            pl.BlockSpec((NC, 1000, d), lambda i: (0, i, 0)),
            pl.BlockSpec((1000, 1), lambda i: (i, 0)),
            pl.BlockSpec((1000, d), lambda i: (i, 0)),
        ],
        out_specs=pl.BlockSpec((1000, d), lambda i: (i, 0)),
        out_shape=jax.ShapeDtypeStruct((e, d), f32),
        input_output_aliases={3: 0},
    )(t1, g1, dinv, out1raw)

    neb = e // 1000
    ab = pl.pallas_call(
        functools.partial(_k_stats, neb, float(e)),
        grid=(neb,),
        in_specs=[
            pl.BlockSpec((1000, d), lambda i: (i, 0)),
            pl.BlockSpec((1, d), lambda i: (0, 0)),
            pl.BlockSpec((1, d), lambda i: (0, 0)),
        ],
        out_specs=pl.BlockSpec((2, d), lambda i: (0, 0)),
        out_shape=jax.ShapeDtypeStruct((2, d), f32),
        scratch_shapes=[pltpu.VMEM((2, d), f32)],
    )(out1, bng2, bnb2)

    t2 = pl.pallas_call(
        _k_h2low,
        grid=(nlb,),
        in_specs=[
            pl.BlockSpec((1000, d), lambda i: (i, 0)),
            pl.BlockSpec((2, d), lambda i: (0, 0)),
            pl.BlockSpec((d, d), lambda i: (0, 0)),
            pl.BlockSpec((1000, 1), lambda i: (i, 0)),
        ],
        out_specs=pl.BlockSpec((1000, d), lambda i: (i, 0)),
        out_shape=jax.ShapeDtypeStruct((n, d), f32),
    )(out1, ab, w2t, dinv)

    nhb = (e - n) // 1000
    hi = pl.pallas_call(
        _k_phase2,
        grid=(nhb,),
        in_specs=[
            pl.BlockSpec((1000, d), lambda i: (i + nlb, 0)),
            pl.BlockSpec((2, d), lambda i: (0, 0)),
            pl.BlockSpec((d, d), lambda i: (0, 0)),
            pl.BlockSpec((1, d), lambda i: (0, 0)),
            pl.BlockSpec((1, d), lambda i: (0, 0)),
            pl.BlockSpec((1, d), lambda i: (0, 0)),
            pl.BlockSpec((d, 1), lambda i: (0, 0)),
            pl.BlockSpec((1, 1), lambda i: (0, 0)),
        ],
        out_specs=pl.BlockSpec((1000, 1), lambda i: (i, 0)),
        out_shape=jax.ShapeDtypeStruct((e - n, 1), f32),
    )(out1, ab, w2t, b22, lng2, lnb2, wlt, bl2)

    g2 = sc_agg(t2, src, dst, zrow)

    low = pl.pallas_call(
        _k_final,
        grid=(nlb,),
        in_specs=[
            pl.BlockSpec((1000, d), lambda i: (i, 0)),
            pl.BlockSpec((NC, 1000, d), lambda i: (0, i, 0)),
            pl.BlockSpec((1000, 1), lambda i: (i, 0)),
            pl.BlockSpec((1, d), lambda i: (0, 0)),
            pl.BlockSpec((1, d), lambda i: (0, 0)),
            pl.BlockSpec((1, d), lambda i: (0, 0)),
            pl.BlockSpec((d, 1), lambda i: (0, 0)),
            pl.BlockSpec((1, 1), lambda i: (0, 0)),
        ],
        out_specs=pl.BlockSpec((1000, 1), lambda i: (i, 0)),
        out_shape=jax.ShapeDtypeStruct((n, 1), f32),
    )(t2, g2, dinv, b22, lng2, lnb2, wlt, bl2)

    return jnp.concatenate([low[:, 0], hi[:, 0]])
